# baseline (speedup 1.0000x reference)
import functools

import jax
import jax.numpy as jnp
from jax import lax
from jax.experimental import pallas as pl
from jax.experimental.pallas import tpu as pltpu
from jax.experimental.pallas import tpu_sc as plsc

D = 64
NC, NS = 2, 16
NW = NC * NS
NBUF = 6


def _sc_gather(table, idxn, b_, h_):
    nblk, blk = idxn.shape
    nbt = b_ // 128
    mblk = nblk // NW
    ring = 8
    nch = mblk * 8
    ngrp = nch // ring
    mesh = plsc.VectorSubcoreMesh(core_axis_name="c", subcore_axis_name="s")

    @functools.partial(
        pl.kernel,
        mesh=mesh,
        compiler_params=pltpu.CompilerParams(use_tc_tiling_on_sc=False),
        out_type=jax.ShapeDtypeStruct((b_, h_ * 128), jnp.float32),
        scratch_types=[
            pltpu.VMEM((mblk, blk), jnp.int32),
            pltpu.VMEM((ring, 128, D), jnp.float32),
        ] + [pltpu.SemaphoreType.DMA] * (2 * ring),
    )
    def k(table_hbm, idx_hbm, out_hbm, idx_v, rows_v, *sems):
        gsems, osems = sems[:ring], sems[ring:]
        wid = lax.axis_index("s") * NC + lax.axis_index("c")
        pltpu.sync_copy(idx_hbm.at[pl.ds(wid * mblk, mblk)], idx_v)

        def odst(j):
            m = wid * mblk + j // 8
            ht = m // nbt
            bt = lax.rem(m, nbt)
            h = ht * 8 + lax.rem(j, 8)
            return out_hbm.at[pl.ds(bt * 128, 128), pl.ds(h * 128, D)]

        def gstart(j, b):
            pltpu.async_copy(
                table_hbm.at[idx_v.at[j // 8, pl.ds(lax.rem(j, 8) * 128, 128)]],
                rows_v.at[b],
                gsems[b],
            )

        def gwait(j, b):
            pltpu.make_async_copy(
                table_hbm.at[idx_v.at[j // 8, pl.ds(lax.rem(j, 8) * 128, 128)]],
                rows_v.at[b],
                gsems[b],
            ).wait()

        def ostart(j, b):
            pltpu.async_copy(rows_v.at[b], odst(j), osems[b])

        def owait(j, b):
            pltpu.make_async_copy(rows_v.at[b], odst(j), osems[b]).wait()

        for b in range(NBUF):
            gstart(b, b)

        def body(g, carry):
            for b in range(ring):
                j = g * ring + b
                gwait(j, b)
                ostart(j, b)
                bn = (b + NBUF) % ring
                jn = j + NBUF
                if b < ring - NBUF:
                    @pl.when(g >= 1)
                    def _():
                        owait(jn - ring, bn)
                        gstart(jn, bn)

                    @pl.when(g < 1)
                    def _():
                        gstart(jn, bn)
                else:
                    @pl.when(g < ngrp - 1)
                    def _():
                        owait(jn - ring, bn)
                        gstart(jn, bn)
            return carry

        lax.fori_loop(0, ngrp, body, 0)

        for b in range(ring):
            owait(nch - ring + b, b)

    return k(table, idxn)


def kernel(variate_ids, variate_embed_weight):
    b, h = variate_ids.shape
    idxn = (
        variate_ids.T.reshape(h // 8, 8, b // 128, 128)
        .transpose(0, 2, 1, 3)
        .reshape((h // 8) * (b // 128), 8 * 128)
        .astype(jnp.int32)
    )
    out = _sc_gather(variate_embed_weight, idxn, b, h)
    return out.reshape(b, h, 128)[:, :, :D]
